# baseline (speedup 1.0000x reference)
import functools

import jax
import jax.numpy as jnp
from jax import lax
from jax.experimental import pallas as pl
from jax.experimental.pallas import tpu as pltpu
from jax.experimental.pallas import tpu_sc as plsc

N = 16384
IN = 128
OUT = 128
MID = 128
S = 8
NS = 16

_BQ = 128
_BM = 128
_GCH = 128


def _proj_body(x_ref, wkT_ref, bk_ref, wvT_ref, bv_ref, xk_ref, xv_ref):
    x = x_ref[...]
    xk_ref[...] = (
        jnp.dot(x, wkT_ref[...], preferred_element_type=jnp.float32) + bk_ref[...]
    )
    xv_ref[...] = (
        jnp.dot(x, wvT_ref[...], preferred_element_type=jnp.float32) + bv_ref[...]
    )


def _proj(x, WkT, bk2, WvT, bv2):
    grid = N // _BQ
    return pl.pallas_call(
        _proj_body,
        grid=(grid,),
        in_specs=[
            pl.BlockSpec((_BQ, IN), lambda i: (i, 0)),
            pl.BlockSpec((IN, MID), lambda i: (0, 0)),
            pl.BlockSpec((1, MID), lambda i: (0, 0)),
            pl.BlockSpec((IN, OUT), lambda i: (0, 0)),
            pl.BlockSpec((1, OUT), lambda i: (0, 0)),
        ],
        out_specs=[
            pl.BlockSpec((_BQ, MID), lambda i: (i, 0)),
            pl.BlockSpec((_BQ, OUT), lambda i: (i, 0)),
        ],
        out_shape=[
            jax.ShapeDtypeStruct((N, MID), jnp.float32),
            jax.ShapeDtypeStruct((N, OUT), jnp.float32),
        ],
    )(x, WkT, bk2, WvT, bv2)


_NSH = 256
_NCH = N // _NSH


def _knn_body(pq_ref, pT_ref, idx_ref, d2_ref, mr_ref, jr_ref):
    BQ = _BQ
    INF = jnp.float32(jnp.inf)
    BIGI = jnp.int32(2**30)

    px = pT_ref[0:1, :, :]
    py = pT_ref[1:2, :, :]
    pz = pT_ref[2:3, :, :]
    dx = pq_ref[:, 0:1][:, :, None] - px
    dy = pq_ref[:, 1:2][:, :, None] - py
    dz = pq_ref[:, 2:3][:, :, None] - pz
    d2_ref[...] = dx * dx + dy * dy + dz * dz

    iota3 = (
        lax.broadcasted_iota(jnp.int32, (1, _NCH, _NSH), 1) * _NSH
        + lax.broadcasted_iota(jnp.int32, (1, _NCH, _NSH), 2)
    )

    def _cond(c):
        k, notdone = c
        return jnp.logical_and(k < NS, notdone == 1)

    def _body(c):
        k, _ = c
        d2 = d2_ref[...]
        M = jnp.min(d2, axis=1)
        rmin = jnp.min(M, axis=1, keepdims=True)

        def _cnt(r, acc):
            return acc + jnp.sum(
                jnp.where(mr_ref[r] < rmin, 1, 0), axis=1, keepdims=True
            )

        cnt = lax.fori_loop(0, k, _cnt, jnp.zeros((BQ, 1), jnp.int32))
        certified = jnp.min(jnp.where(cnt >= NS, 1, 0))
        cand = jnp.where(d2 == M[:, None, :], iota3, BIGI)
        J = jnp.min(cand, axis=1)
        d2_ref[...] = jnp.where(iota3 == J[:, None, :], INF, d2)
        mr_ref[k] = M
        jr_ref[k] = J
        return k + 1, jnp.where(certified == 1, 0, 1).astype(jnp.int32)

    kf, _ = lax.while_loop(_cond, _body, (jnp.int32(0), jnp.int32(1)))

    for t in range(NS):
        m = lax.fori_loop(
            0, kf,
            lambda r, acc: jnp.minimum(
                acc, jnp.min(mr_ref[r], axis=1, keepdims=True)
            ),
            jnp.full((BQ, 1), INF),
        )

        def _jmin(r, acc, m=m):
            c = jnp.min(
                jnp.where(mr_ref[r] == m, jr_ref[r], BIGI), axis=1,
                keepdims=True,
            )
            return jnp.minimum(acc, c)

        j = lax.fori_loop(0, kf, _jmin, jnp.full((BQ, 1), BIGI))

        def _mask(r, acc, j=j):
            mr_ref[r] = jnp.where(jr_ref[r] == j, INF, mr_ref[r])
            return acc

        lax.fori_loop(0, kf, _mask, 0)
        idx_ref[:, t : t + 1] = j


def _knn(p8, pT3):
    grid = N // _BQ
    return pl.pallas_call(
        _knn_body,
        grid=(grid,),
        in_specs=[
            pl.BlockSpec((_BQ, 8), lambda i: (i, 0)),
            pl.BlockSpec((8, _NCH, _NSH), lambda i: (0, 0, 0)),
        ],
        out_specs=pl.BlockSpec((_BQ, NS), lambda i: (i, 0)),
        out_shape=jax.ShapeDtypeStruct((N, NS), jnp.int32),
        scratch_shapes=[
            pltpu.VMEM((_BQ, _NCH, _NSH), jnp.float32),
            pltpu.VMEM((NS, _BQ, _NSH), jnp.float32),
            pltpu.VMEM((NS, _BQ, _NSH), jnp.int32),
        ],
    )(p8, pT3)


@functools.cache
def _make_gather():
    nc, nsc = 2, 16
    nw = nc * nsc
    B = N * NS
    b_per_w = B // nw
    n_ch = b_per_w // _GCH
    mesh = plsc.VectorSubcoreMesh(core_axis_name="c", subcore_axis_name="s")

    @functools.partial(
        pl.kernel,
        mesh=mesh,
        out_type=[
            jax.ShapeDtypeStruct((B, MID), jnp.float32),
            jax.ShapeDtypeStruct((B, OUT), jnp.float32),
            jax.ShapeDtypeStruct((B, 128), jnp.float32),
        ],
        scratch_types=[
            pltpu.VMEM((_GCH,), jnp.int32),
            pltpu.VMEM((_GCH, MID), jnp.float32),
            pltpu.VMEM((_GCH, OUT), jnp.float32),
            pltpu.VMEM((_GCH, 128), jnp.float32),
            pltpu.SemaphoreType.DMA,
            pltpu.SemaphoreType.DMA,
            pltpu.SemaphoreType.DMA,
        ],
    )
    def gather_k(kt_hbm, vt_hbm, p128_hbm, idx_hbm,
                 xk_hbm, xv_hbm, pg_hbm,
                 idx_v, kv, vv, pv, sem1, sem2, sem3):
        wid = lax.axis_index("s") * nc + lax.axis_index("c")
        base = wid * b_per_w

        def body(i, carry):
            off = base + i * _GCH
            pltpu.sync_copy(idx_hbm.at[pl.ds(off, _GCH)], idx_v)
            c1 = pltpu.async_copy(kt_hbm.at[idx_v], kv, sem1)
            c2 = pltpu.async_copy(vt_hbm.at[idx_v], vv, sem2)
            c3 = pltpu.async_copy(p128_hbm.at[idx_v], pv, sem3)
            c1.wait()
            c2.wait()
            c3.wait()
            pltpu.sync_copy(kv, xk_hbm.at[pl.ds(off, _GCH)])
            pltpu.sync_copy(vv, xv_hbm.at[pl.ds(off, _GCH)])
            pltpu.sync_copy(pv, pg_hbm.at[pl.ds(off, _GCH)])
            return carry

        lax.fori_loop(0, n_ch, body, 0)

    return gather_k


def _mlp_body(x_ref, pq_ref, xkg_ref, xvg_ref, pg_ref,
              wqT_ref, bq_ref, sm_ref, wp2T_ref, bp2_ref,
              g1_ref, beta1_ref, wl1T_ref, bl1_ref,
              g2_ref, beta2_ref, wl2T_ref, bl2_ref, out_ref):
    BM = _BM
    R = BM * NS

    xq = jnp.dot(x_ref[...], wqT_ref[...], preferred_element_type=jnp.float32)
    xq = xq + bq_ref[...]
    xq_rep = jnp.broadcast_to(xq[:, None, :], (BM, NS, MID)).reshape(R, MID)

    pq_rep = jnp.broadcast_to(pq_ref[...][:, None, :], (BM, NS, 16)).reshape(R, 16)
    p_r = pg_ref[:, 0:16] - pq_rep

    cols = []
    for jj in range(3):
        acc = sm_ref[9 + jj]
        for kk in range(3):
            acc = acc + p_r[:, kk : kk + 1] * sm_ref[3 * jj + kk]
        acc = jnp.maximum(acc * sm_ref[12 + jj] + sm_ref[15 + jj], 0.0)
        cols.append(acc)
    pr2 = bp2_ref[...]
    for jj in range(3):
        pr2 = pr2 + cols[jj] * wp2T_ref[jj : jj + 1, :]

    w = xkg_ref[...] - xq_rep + pr2
    w = jnp.maximum(w * g1_ref[...] + beta1_ref[...], 0.0)
    w1 = jnp.dot(w, wl1T_ref[...], preferred_element_type=jnp.float32) + bl1_ref[...]
    w1 = jnp.maximum(w1 * g2_ref[...] + beta2_ref[...], 0.0)
    w2 = jnp.dot(w1, wl2T_ref[...], preferred_element_type=jnp.float32) + bl2_ref[...]

    w3 = w2.reshape(BM, NS, MID // S)
    mx = jnp.max(w3, axis=1, keepdims=True)
    e = jnp.exp(w3 - mx)
    ssum = jnp.sum(e, axis=1, keepdims=True)
    smx = (e / ssum).reshape(R, MID // S)
    wfull = jnp.concatenate([smx] * S, axis=1)

    v = (xvg_ref[...] + pr2) * wfull
    out_ref[...] = v.reshape(BM, NS, OUT).sum(axis=1)


def _mlp(x, p16, xkg, xvg, pg, wqT, bq2, smalls, wp2T, bp2r,
         g1r, beta1r, wl1T, bl1r, g2r, beta2r, wl2T, bl2r):
    grid = N // _BM
    full = lambda shape: pl.BlockSpec(shape, lambda i: tuple(0 for _ in shape))
    return pl.pallas_call(
        _mlp_body,
        grid=(grid,),
        in_specs=[
            pl.BlockSpec((_BM, IN), lambda i: (i, 0)),
            pl.BlockSpec((_BM, 16), lambda i: (i, 0)),
            pl.BlockSpec((_BM * NS, MID), lambda i: (i, 0)),
            pl.BlockSpec((_BM * NS, OUT), lambda i: (i, 0)),
            pl.BlockSpec((_BM * NS, 128), lambda i: (i, 0)),
            full((IN, MID)),
            full((1, MID)),
            pl.BlockSpec(memory_space=pltpu.SMEM),
            full((8, OUT)),
            full((1, OUT)),
            full((1, MID)),
            full((1, MID)),
            full((MID, MID // S)),
            full((1, MID // S)),
            full((1, MID // S)),
            full((1, MID // S)),
            full((MID // S, MID // S)),
            full((1, MID // S)),
        ],
        out_specs=pl.BlockSpec((_BM, OUT), lambda i: (i, 0)),
        out_shape=jax.ShapeDtypeStruct((N, OUT), jnp.float32),
    )(x, p16, xkg, xvg, pg, wqT, bq2, smalls, wp2T, bp2r,
      g1r, beta1r, wl1T, bl1r, g2r, beta2r, wl2T, bl2r)


def kernel(p, x, o, Wq, bq, Wk, bk, Wv, bv, Wp1, bp1, gp, betap, Wp2, bp2,
           g1, beta1, Wl1, bl1, g2, beta2, Wl2, bl2):
    f32 = jnp.float32
    bnscale = 1.0 / jnp.sqrt(jnp.float32(1.0 + 1e-5))

    p8 = jnp.zeros((N, 8), f32).at[:, :3].set(p)
    pT8 = jnp.zeros((8, N), f32).at[:3, :].set(p.T)

    smalls = jnp.concatenate([
        Wp1.reshape(-1),
        bp1,
        gp * bnscale,
        betap,
        jnp.zeros((14,), f32),
    ])

    xk0, xv0 = _proj(x, Wk.T, bk[None, :], Wv.T, bv[None, :])
    idx = _knn(p8, pT8.reshape(8, _NCH, _NSH))
    idxf = idx.reshape(-1)
    p128 = jnp.zeros((N, 128), f32).at[:, :3].set(p)
    p16 = p128[:, :16]
    xkg, xvg, pg = _make_gather()(xk0, xv0, p128, idxf)

    out = _mlp(
        x, p16, xkg, xvg, pg,
        Wq.T, bq[None, :], smalls,
        jnp.zeros((8, OUT), f32).at[:3, :].set(Wp2.T), bp2[None, :],
        (g1 * bnscale)[None, :], beta1[None, :],
        Wl1.T, bl1[None, :],
        (g2 * bnscale)[None, :], beta2[None, :],
        Wl2.T, bl2[None, :],
    )
    return out


# speedup vs baseline: 1.3316x; 1.3316x over previous
import functools

import jax
import jax.numpy as jnp
from jax import lax
from jax.experimental import pallas as pl
from jax.experimental.pallas import tpu as pltpu
from jax.experimental.pallas import tpu_sc as plsc

N = 16384
IN = 128
OUT = 128
MID = 128
S = 8
NS = 16

_BQ = 256
_BM = 256
_GCH = 128


def _proj_body(x_ref, wkT_ref, bk_ref, wvT_ref, bv_ref, xk_ref, xv_ref):
    x = x_ref[...]
    xk_ref[...] = (
        jnp.dot(x, wkT_ref[...], preferred_element_type=jnp.float32) + bk_ref[...]
    )
    xv_ref[...] = (
        jnp.dot(x, wvT_ref[...], preferred_element_type=jnp.float32) + bv_ref[...]
    )


def _proj(x, WkT, bk2, WvT, bv2):
    grid = N // _BQ
    return pl.pallas_call(
        _proj_body,
        grid=(grid,),
        in_specs=[
            pl.BlockSpec((_BQ, IN), lambda i: (i, 0)),
            pl.BlockSpec((IN, MID), lambda i: (0, 0)),
            pl.BlockSpec((1, MID), lambda i: (0, 0)),
            pl.BlockSpec((IN, OUT), lambda i: (0, 0)),
            pl.BlockSpec((1, OUT), lambda i: (0, 0)),
        ],
        out_specs=[
            pl.BlockSpec((_BQ, MID), lambda i: (i, 0)),
            pl.BlockSpec((_BQ, OUT), lambda i: (i, 0)),
        ],
        out_shape=[
            jax.ShapeDtypeStruct((N, MID), jnp.float32),
            jax.ShapeDtypeStruct((N, OUT), jnp.float32),
        ],
    )(x, WkT, bk2, WvT, bv2)


_NSH = 256
_NCH = N // _NSH


def _knn_body(pq_ref, pT_ref, idx_ref, d2_ref, rv_ref, ri_ref):
    BQ = _BQ
    INF = jnp.float32(jnp.inf)
    BIGI = jnp.int32(2**30)

    px = pT_ref[0:1, :, :]
    py = pT_ref[1:2, :, :]
    pz = pT_ref[2:3, :, :]
    dx = pq_ref[:, 0:1][:, :, None] - px
    dy = pq_ref[:, 1:2][:, :, None] - py
    dz = pq_ref[:, 2:3][:, :, None] - pz
    d2_ref[...] = dx * dx + dy * dy + dz * dz

    iota3 = (
        lax.broadcasted_iota(jnp.int32, (1, _NCH, _NSH), 1) * _NSH
        + lax.broadcasted_iota(jnp.int32, (1, _NCH, _NSH), 2)
    )
    rv_ref[...] = jnp.full((BQ, NS), INF)
    ri_ref[...] = jnp.zeros((BQ, NS), jnp.int32)

    def _cond(c):
        k, notdone = c
        return jnp.logical_and(k < NS, notdone == 1)

    def _body(c):
        k, _ = c
        d2 = d2_ref[...]
        M = jnp.min(d2, axis=1)
        rmin = jnp.min(M, axis=1, keepdims=True)
        cnt = jnp.sum(
            jnp.where(rv_ref[...] < rmin, 1, 0), axis=1, keepdims=True
        )
        certified = jnp.min(jnp.where(cnt >= NS, 1, 0))
        cand = jnp.where(d2 == M[:, None, :], iota3, BIGI)
        J = jnp.min(cand, axis=1)
        d2_ref[...] = jnp.where(iota3 == J[:, None, :], INF, d2)
        pv = jnp.concatenate([rv_ref[...], M], axis=1)
        pi = jnp.concatenate([ri_ref[...], J], axis=1)
        nv, ni = [], []
        for _t in range(NS):
            m = jnp.min(pv, axis=1, keepdims=True)
            j = jnp.min(jnp.where(pv == m, pi, BIGI), axis=1, keepdims=True)
            pv = jnp.where(pi == j, INF, pv)
            nv.append(m)
            ni.append(j)
        rv_ref[...] = jnp.concatenate(nv, axis=1)
        ri_ref[...] = jnp.concatenate(ni, axis=1)
        return k + 1, jnp.where(certified == 1, 0, 1).astype(jnp.int32)

    lax.while_loop(_cond, _body, (jnp.int32(0), jnp.int32(1)))

    idx_ref[...] = ri_ref[...]


def _knn(p8, pT3):
    grid = N // _BQ
    return pl.pallas_call(
        _knn_body,
        grid=(grid,),
        in_specs=[
            pl.BlockSpec((_BQ, 8), lambda i: (i, 0)),
            pl.BlockSpec((8, _NCH, _NSH), lambda i: (0, 0, 0)),
        ],
        out_specs=pl.BlockSpec((_BQ, NS), lambda i: (i, 0)),
        out_shape=jax.ShapeDtypeStruct((N, NS), jnp.int32),
        scratch_shapes=[
            pltpu.VMEM((_BQ, _NCH, _NSH), jnp.float32),
            pltpu.VMEM((_BQ, NS), jnp.float32),
            pltpu.VMEM((_BQ, NS), jnp.int32),
        ],
    )(p8, pT3)


@functools.cache
def _make_gather():
    nc, nsc = 2, 16
    nw = nc * nsc
    B = N * NS
    b_per_w = B // nw
    n_ch = b_per_w // _GCH
    mesh = plsc.VectorSubcoreMesh(core_axis_name="c", subcore_axis_name="s")

    @functools.partial(
        pl.kernel,
        mesh=mesh,
        out_type=[
            jax.ShapeDtypeStruct((B, MID), jnp.float32),
            jax.ShapeDtypeStruct((B, OUT), jnp.float32),
            jax.ShapeDtypeStruct((B, 128), jnp.float32),
        ],
        scratch_types=[
            pltpu.VMEM((_GCH,), jnp.int32),
            pltpu.VMEM((_GCH, MID), jnp.float32),
            pltpu.VMEM((_GCH, OUT), jnp.float32),
            pltpu.VMEM((_GCH, 128), jnp.float32),
            pltpu.SemaphoreType.DMA,
            pltpu.SemaphoreType.DMA,
            pltpu.SemaphoreType.DMA,
        ],
    )
    def gather_k(kt_hbm, vt_hbm, p128_hbm, idx_hbm,
                 xk_hbm, xv_hbm, pg_hbm,
                 idx_v, kv, vv, pv, sem1, sem2, sem3):
        wid = lax.axis_index("s") * nc + lax.axis_index("c")
        base = wid * b_per_w

        def body(i, carry):
            off = base + i * _GCH
            pltpu.sync_copy(idx_hbm.at[pl.ds(off, _GCH)], idx_v)
            c1 = pltpu.async_copy(kt_hbm.at[idx_v], kv, sem1)
            c2 = pltpu.async_copy(vt_hbm.at[idx_v], vv, sem2)
            c3 = pltpu.async_copy(p128_hbm.at[idx_v], pv, sem3)
            c1.wait()
            c2.wait()
            c3.wait()
            pltpu.sync_copy(kv, xk_hbm.at[pl.ds(off, _GCH)])
            pltpu.sync_copy(vv, xv_hbm.at[pl.ds(off, _GCH)])
            pltpu.sync_copy(pv, pg_hbm.at[pl.ds(off, _GCH)])
            return carry

        lax.fori_loop(0, n_ch, body, 0)

    return gather_k


def _mlp_body(x_ref, pq_ref, xkg_ref, xvg_ref, pg_ref,
              wqT_ref, bq_ref, sm_ref, wp2T_ref, bp2_ref,
              g1_ref, beta1_ref, wl1T_ref, bl1_ref,
              g2_ref, beta2_ref, wl2T_ref, bl2_ref, out_ref):
    BM = _BM
    R = BM * NS

    xq = jnp.dot(x_ref[...], wqT_ref[...], preferred_element_type=jnp.float32)
    xq = xq + bq_ref[...]
    xq_rep = jnp.broadcast_to(xq[:, None, :], (BM, NS, MID)).reshape(R, MID)

    pq_rep = jnp.broadcast_to(pq_ref[...][:, None, :], (BM, NS, 16)).reshape(R, 16)
    p_r = pg_ref[:, 0:16] - pq_rep

    cols = []
    for jj in range(3):
        acc = sm_ref[9 + jj]
        for kk in range(3):
            acc = acc + p_r[:, kk : kk + 1] * sm_ref[3 * jj + kk]
        acc = jnp.maximum(acc * sm_ref[12 + jj] + sm_ref[15 + jj], 0.0)
        cols.append(acc)
    pr2 = bp2_ref[...]
    for jj in range(3):
        pr2 = pr2 + cols[jj] * wp2T_ref[jj : jj + 1, :]

    w = xkg_ref[...] - xq_rep + pr2
    w = jnp.maximum(w * g1_ref[...] + beta1_ref[...], 0.0)
    w1 = jnp.dot(w, wl1T_ref[...], preferred_element_type=jnp.float32) + bl1_ref[...]
    w1 = jnp.maximum(w1 * g2_ref[...] + beta2_ref[...], 0.0)
    w2 = jnp.dot(w1, wl2T_ref[...], preferred_element_type=jnp.float32) + bl2_ref[...]

    w3 = w2.reshape(BM, NS, MID // S)
    mx = jnp.max(w3, axis=1, keepdims=True)
    e = jnp.exp(w3 - mx)
    ssum = jnp.sum(e, axis=1, keepdims=True)
    smx = (e / ssum).reshape(R, MID // S)
    wfull = jnp.concatenate([smx] * S, axis=1)

    v = (xvg_ref[...] + pr2) * wfull
    out_ref[...] = v.reshape(BM, NS, OUT).sum(axis=1)


def _mlp(x, p16, xkg, xvg, pg, wqT, bq2, smalls, wp2T, bp2r,
         g1r, beta1r, wl1T, bl1r, g2r, beta2r, wl2T, bl2r):
    grid = N // _BM
    full = lambda shape: pl.BlockSpec(shape, lambda i: tuple(0 for _ in shape))
    return pl.pallas_call(
        _mlp_body,
        grid=(grid,),
        in_specs=[
            pl.BlockSpec((_BM, IN), lambda i: (i, 0)),
            pl.BlockSpec((_BM, 16), lambda i: (i, 0)),
            pl.BlockSpec((_BM * NS, MID), lambda i: (i, 0)),
            pl.BlockSpec((_BM * NS, OUT), lambda i: (i, 0)),
            pl.BlockSpec((_BM * NS, 128), lambda i: (i, 0)),
            full((IN, MID)),
            full((1, MID)),
            pl.BlockSpec(memory_space=pltpu.SMEM),
            full((8, OUT)),
            full((1, OUT)),
            full((1, MID)),
            full((1, MID)),
            full((MID, MID // S)),
            full((1, MID // S)),
            full((1, MID // S)),
            full((1, MID // S)),
            full((MID // S, MID // S)),
            full((1, MID // S)),
        ],
        out_specs=pl.BlockSpec((_BM, OUT), lambda i: (i, 0)),
        out_shape=jax.ShapeDtypeStruct((N, OUT), jnp.float32),
    )(x, p16, xkg, xvg, pg, wqT, bq2, smalls, wp2T, bp2r,
      g1r, beta1r, wl1T, bl1r, g2r, beta2r, wl2T, bl2r)


def kernel(p, x, o, Wq, bq, Wk, bk, Wv, bv, Wp1, bp1, gp, betap, Wp2, bp2,
           g1, beta1, Wl1, bl1, g2, beta2, Wl2, bl2):
    f32 = jnp.float32
    bnscale = 1.0 / jnp.sqrt(jnp.float32(1.0 + 1e-5))

    p8 = jnp.zeros((N, 8), f32).at[:, :3].set(p)
    pT8 = jnp.zeros((8, N), f32).at[:3, :].set(p.T)

    smalls = jnp.concatenate([
        Wp1.reshape(-1),
        bp1,
        gp * bnscale,
        betap,
        jnp.zeros((14,), f32),
    ])

    xk0, xv0 = _proj(x, Wk.T, bk[None, :], Wv.T, bv[None, :])
    idx = _knn(p8, pT8.reshape(8, _NCH, _NSH))
    idxf = idx.reshape(-1)
    p128 = jnp.zeros((N, 128), f32).at[:, :3].set(p)
    p16 = p128[:, :16]
    xkg, xvg, pg = _make_gather()(xk0, xv0, p128, idxf)

    out = _mlp(
        x, p16, xkg, xvg, pg,
        Wq.T, bq[None, :], smalls,
        jnp.zeros((8, OUT), f32).at[:3, :].set(Wp2.T), bp2[None, :],
        (g1 * bnscale)[None, :], beta1[None, :],
        Wl1.T, bl1[None, :],
        (g2 * bnscale)[None, :], beta2[None, :],
        Wl2.T, bl2[None, :],
    )
    return out


# speedup vs baseline: 1.3525x; 1.0157x over previous
import functools

import jax
import jax.numpy as jnp
from jax import lax
from jax.experimental import pallas as pl
from jax.experimental.pallas import tpu as pltpu
from jax.experimental.pallas import tpu_sc as plsc

N = 16384
IN = 128
OUT = 128
MID = 128
S = 8
NS = 16

_BQ = 256
_BM = 256
_GCH = 128


def _proj_body(x_ref, wkT_ref, bk_ref, wvT_ref, bv_ref, xk_ref, xv_ref):
    x = x_ref[...]
    xk_ref[...] = (
        jnp.dot(x, wkT_ref[...], preferred_element_type=jnp.float32) + bk_ref[...]
    )
    xv_ref[...] = (
        jnp.dot(x, wvT_ref[...], preferred_element_type=jnp.float32) + bv_ref[...]
    )


def _proj(x, WkT, bk2, WvT, bv2):
    grid = N // _BQ
    return pl.pallas_call(
        _proj_body,
        grid=(grid,),
        in_specs=[
            pl.BlockSpec((_BQ, IN), lambda i: (i, 0)),
            pl.BlockSpec((IN, MID), lambda i: (0, 0)),
            pl.BlockSpec((1, MID), lambda i: (0, 0)),
            pl.BlockSpec((IN, OUT), lambda i: (0, 0)),
            pl.BlockSpec((1, OUT), lambda i: (0, 0)),
        ],
        out_specs=[
            pl.BlockSpec((_BQ, MID), lambda i: (i, 0)),
            pl.BlockSpec((_BQ, OUT), lambda i: (i, 0)),
        ],
        out_shape=[
            jax.ShapeDtypeStruct((N, MID), jnp.float32),
            jax.ShapeDtypeStruct((N, OUT), jnp.float32),
        ],
    )(x, WkT, bk2, WvT, bv2)


_NSH = 256
_NCH = N // _NSH


def _knn_body(pq_ref, pT_ref, idx_ref, d2_ref, rv_ref, ri_ref):
    BQ = _BQ
    INF = jnp.float32(jnp.inf)
    BIGI = jnp.int32(2**30)

    px = pT_ref[0:1, :, :]
    py = pT_ref[1:2, :, :]
    pz = pT_ref[2:3, :, :]
    dx = pq_ref[:, 0:1][:, :, None] - px
    dy = pq_ref[:, 1:2][:, :, None] - py
    dz = pq_ref[:, 2:3][:, :, None] - pz
    d2_ref[...] = dx * dx + dy * dy + dz * dz

    iota3 = (
        lax.broadcasted_iota(jnp.int32, (1, _NCH, _NSH), 1) * _NSH
        + lax.broadcasted_iota(jnp.int32, (1, _NCH, _NSH), 2)
    )
    rv_ref[...] = jnp.full((BQ, NS), INF)
    ri_ref[...] = jnp.zeros((BQ, NS), jnp.int32)

    def _cond(c):
        k, notdone = c
        return jnp.logical_and(k < NS, notdone == 1)

    def _body(c):
        k, _ = c
        d2 = d2_ref[...]
        M = jnp.min(d2, axis=1)
        rmin = jnp.min(M, axis=1, keepdims=True)
        cnt = jnp.sum(
            jnp.where(rv_ref[...] < rmin, 1, 0), axis=1, keepdims=True
        )
        certified = jnp.min(jnp.where(cnt >= NS, 1, 0))
        cand = jnp.where(d2 == M[:, None, :], iota3, BIGI)
        J = jnp.min(cand, axis=1)
        d2_ref[...] = jnp.where(iota3 == J[:, None, :], INF, d2)
        pv = jnp.concatenate([rv_ref[...], M], axis=1)
        pi = jnp.concatenate([ri_ref[...], J], axis=1)
        nv, ni = [], []
        for _t in range(NS):
            m = jnp.min(pv, axis=1, keepdims=True)
            j = jnp.min(jnp.where(pv == m, pi, BIGI), axis=1, keepdims=True)
            pv = jnp.where(pi == j, INF, pv)
            nv.append(m)
            ni.append(j)
        rv_ref[...] = jnp.concatenate(nv, axis=1)
        ri_ref[...] = jnp.concatenate(ni, axis=1)
        return k + 1, jnp.where(certified == 1, 0, 1).astype(jnp.int32)

    lax.while_loop(_cond, _body, (jnp.int32(0), jnp.int32(1)))

    idx_ref[...] = ri_ref[...]


def _knn(p8, pT3):
    grid = N // _BQ
    return pl.pallas_call(
        _knn_body,
        grid=(grid,),
        in_specs=[
            pl.BlockSpec((_BQ, 8), lambda i: (i, 0)),
            pl.BlockSpec((8, _NCH, _NSH), lambda i: (0, 0, 0)),
        ],
        out_specs=pl.BlockSpec((_BQ, NS), lambda i: (i, 0)),
        out_shape=jax.ShapeDtypeStruct((N, NS), jnp.int32),
        scratch_shapes=[
            pltpu.VMEM((_BQ, _NCH, _NSH), jnp.float32),
            pltpu.VMEM((_BQ, NS), jnp.float32),
            pltpu.VMEM((_BQ, NS), jnp.int32),
        ],
    )(p8, pT3)


@functools.cache
def _make_gather():
    nc, nsc = 2, 16
    nw = nc * nsc
    B = N * NS
    b_per_w = B // nw
    n_ch = b_per_w // _GCH
    mesh = plsc.VectorSubcoreMesh(core_axis_name="c", subcore_axis_name="s")

    @functools.partial(
        pl.kernel,
        mesh=mesh,
        out_type=[
            jax.ShapeDtypeStruct((B, MID), jnp.float32),
            jax.ShapeDtypeStruct((B, OUT), jnp.float32),
            jax.ShapeDtypeStruct((B, 128), jnp.float32),
        ],
        scratch_types=[
            pltpu.VMEM((_GCH,), jnp.int32),
            pltpu.VMEM((_GCH,), jnp.int32),
            pltpu.VMEM((_GCH, MID), jnp.float32),
            pltpu.VMEM((_GCH, OUT), jnp.float32),
            pltpu.VMEM((_GCH, 128), jnp.float32),
            pltpu.VMEM((_GCH, MID), jnp.float32),
            pltpu.VMEM((_GCH, OUT), jnp.float32),
            pltpu.VMEM((_GCH, 128), jnp.float32),
            pltpu.SemaphoreType.DMA,
            pltpu.SemaphoreType.DMA,
            pltpu.SemaphoreType.DMA,
            pltpu.SemaphoreType.DMA,
        ],
    )
    def gather_k(kt_hbm, vt_hbm, p128_hbm, idx_hbm,
                 xk_hbm, xv_hbm, pg_hbm,
                 idxa, idxb, kva, vva, pva, kvb, vvb, pvb,
                 sga, sgb, soa, sob):
        wid = lax.axis_index("s") * nc + lax.axis_index("c")
        base = wid * b_per_w

        def body(g, carry):
            offa = base + (2 * g) * _GCH
            offb = offa + _GCH
            pltpu.sync_copy(idx_hbm.at[pl.ds(offa, _GCH)], idxa)
            a1 = pltpu.async_copy(kt_hbm.at[idxa], kva, sga)
            a2 = pltpu.async_copy(vt_hbm.at[idxa], vva, sga)
            a3 = pltpu.async_copy(p128_hbm.at[idxa], pva, sga)
            pltpu.sync_copy(idx_hbm.at[pl.ds(offb, _GCH)], idxb)
            b1 = pltpu.async_copy(kt_hbm.at[idxb], kvb, sgb)
            b2 = pltpu.async_copy(vt_hbm.at[idxb], vvb, sgb)
            b3 = pltpu.async_copy(p128_hbm.at[idxb], pvb, sgb)
            a1.wait()
            a2.wait()
            a3.wait()
            oa1 = pltpu.async_copy(kva, xk_hbm.at[pl.ds(offa, _GCH)], soa)
            oa2 = pltpu.async_copy(vva, xv_hbm.at[pl.ds(offa, _GCH)], soa)
            oa3 = pltpu.async_copy(pva, pg_hbm.at[pl.ds(offa, _GCH)], soa)
            b1.wait()
            b2.wait()
            b3.wait()
            ob1 = pltpu.async_copy(kvb, xk_hbm.at[pl.ds(offb, _GCH)], sob)
            ob2 = pltpu.async_copy(vvb, xv_hbm.at[pl.ds(offb, _GCH)], sob)
            ob3 = pltpu.async_copy(pvb, pg_hbm.at[pl.ds(offb, _GCH)], sob)
            oa1.wait()
            oa2.wait()
            oa3.wait()
            ob1.wait()
            ob2.wait()
            ob3.wait()
            return carry

        lax.fori_loop(0, n_ch // 2, body, 0)

    return gather_k


def _mlp_body(x_ref, pq_ref, xkg_ref, xvg_ref, pg_ref,
              wqT_ref, bq_ref, sm_ref, wp2T_ref, bp2_ref,
              g1_ref, beta1_ref, wl1T_ref, bl1_ref,
              g2_ref, beta2_ref, wl2T_ref, bl2_ref, out_ref):
    BM = _BM
    R = BM * NS

    xq = jnp.dot(x_ref[...], wqT_ref[...], preferred_element_type=jnp.float32)
    xq = xq + bq_ref[...]
    xq_rep = jnp.broadcast_to(xq[:, None, :], (BM, NS, MID)).reshape(R, MID)

    pq_rep = jnp.broadcast_to(pq_ref[...][:, None, :], (BM, NS, 16)).reshape(R, 16)
    p_r = pg_ref[:, 0:16] - pq_rep

    cols = []
    for jj in range(3):
        acc = sm_ref[9 + jj]
        for kk in range(3):
            acc = acc + p_r[:, kk : kk + 1] * sm_ref[3 * jj + kk]
        acc = jnp.maximum(acc * sm_ref[12 + jj] + sm_ref[15 + jj], 0.0)
        cols.append(acc)
    pr2 = bp2_ref[...]
    for jj in range(3):
        pr2 = pr2 + cols[jj] * wp2T_ref[jj : jj + 1, :]

    w = xkg_ref[...] - xq_rep + pr2
    w = jnp.maximum(w * g1_ref[...] + beta1_ref[...], 0.0)
    w1 = jnp.dot(w, wl1T_ref[...], preferred_element_type=jnp.float32) + bl1_ref[...]
    w1 = jnp.maximum(w1 * g2_ref[...] + beta2_ref[...], 0.0)
    w2 = jnp.dot(w1, wl2T_ref[...], preferred_element_type=jnp.float32) + bl2_ref[...]

    w3 = w2.reshape(BM, NS, MID // S)
    mx = jnp.max(w3, axis=1, keepdims=True)
    e = jnp.exp(w3 - mx)
    ssum = jnp.sum(e, axis=1, keepdims=True)
    smx = (e / ssum).reshape(R, MID // S)
    wfull = jnp.concatenate([smx] * S, axis=1)

    v = (xvg_ref[...] + pr2) * wfull
    out_ref[...] = v.reshape(BM, NS, OUT).sum(axis=1)


def _mlp(x, p16, xkg, xvg, pg, wqT, bq2, smalls, wp2T, bp2r,
         g1r, beta1r, wl1T, bl1r, g2r, beta2r, wl2T, bl2r):
    grid = N // _BM
    full = lambda shape: pl.BlockSpec(shape, lambda i: tuple(0 for _ in shape))
    return pl.pallas_call(
        _mlp_body,
        grid=(grid,),
        in_specs=[
            pl.BlockSpec((_BM, IN), lambda i: (i, 0)),
            pl.BlockSpec((_BM, 16), lambda i: (i, 0)),
            pl.BlockSpec((_BM * NS, MID), lambda i: (i, 0)),
            pl.BlockSpec((_BM * NS, OUT), lambda i: (i, 0)),
            pl.BlockSpec((_BM * NS, 128), lambda i: (i, 0)),
            full((IN, MID)),
            full((1, MID)),
            pl.BlockSpec(memory_space=pltpu.SMEM),
            full((8, OUT)),
            full((1, OUT)),
            full((1, MID)),
            full((1, MID)),
            full((MID, MID // S)),
            full((1, MID // S)),
            full((1, MID // S)),
            full((1, MID // S)),
            full((MID // S, MID // S)),
            full((1, MID // S)),
        ],
        out_specs=pl.BlockSpec((_BM, OUT), lambda i: (i, 0)),
        out_shape=jax.ShapeDtypeStruct((N, OUT), jnp.float32),
    )(x, p16, xkg, xvg, pg, wqT, bq2, smalls, wp2T, bp2r,
      g1r, beta1r, wl1T, bl1r, g2r, beta2r, wl2T, bl2r)


def kernel(p, x, o, Wq, bq, Wk, bk, Wv, bv, Wp1, bp1, gp, betap, Wp2, bp2,
           g1, beta1, Wl1, bl1, g2, beta2, Wl2, bl2):
    f32 = jnp.float32
    bnscale = 1.0 / jnp.sqrt(jnp.float32(1.0 + 1e-5))

    p8 = jnp.zeros((N, 8), f32).at[:, :3].set(p)
    pT8 = jnp.zeros((8, N), f32).at[:3, :].set(p.T)

    smalls = jnp.concatenate([
        Wp1.reshape(-1),
        bp1,
        gp * bnscale,
        betap,
        jnp.zeros((14,), f32),
    ])

    xk0, xv0 = _proj(x, Wk.T, bk[None, :], Wv.T, bv[None, :])
    idx = _knn(p8, pT8.reshape(8, _NCH, _NSH))
    idxf = idx.reshape(-1)
    p128 = jnp.zeros((N, 128), f32).at[:, :3].set(p)
    p16 = p128[:, :16]
    xkg, xvg, pg = _make_gather()(xk0, xv0, p128, idxf)

    out = _mlp(
        x, p16, xkg, xvg, pg,
        Wq.T, bq[None, :], smalls,
        jnp.zeros((8, OUT), f32).at[:3, :].set(Wp2.T), bp2[None, :],
        (g1 * bnscale)[None, :], beta1[None, :],
        Wl1.T, bl1[None, :],
        (g2 * bnscale)[None, :], beta2[None, :],
        Wl2.T, bl2[None, :],
    )
    return out
